# baseline (speedup 1.0000x reference)
import functools

import jax
import jax.numpy as jnp
from jax import lax
from jax.experimental import pallas as pl
from jax.experimental.pallas import tpu as pltpu
from jax.experimental.pallas import tpu_sc as plsc

_N_ROWS = 819200
_TABLE_ROWS = 1000000
_D = 64
_NW = 32
_T = 256
_CPW = _N_ROWS // (_NW * _T)
_PAIRS = _CPW // 2
_KH = _T // 128
_NKH = _N_ROWS // 128

_mesh = plsc.VectorSubcoreMesh(core_axis_name="c", subcore_axis_name="s")


@functools.partial(
    pl.kernel,
    out_type=jax.ShapeDtypeStruct((8 * _NKH * 8, 128), jnp.float32),
    mesh=_mesh,
    scratch_types=[
        pltpu.VMEM((_CPW, _T), jnp.int32),
        pltpu.VMEM((_T, _D), jnp.float32),
        pltpu.VMEM((_T, _D), jnp.float32),
        pltpu.VMEM((8 * _KH * 8, 128), jnp.float32),
        pltpu.VMEM((8 * _KH * 8, 128), jnp.float32),
        pltpu.SemaphoreType.DMA,
        pltpu.SemaphoreType.DMA,
        pltpu.SemaphoreType.DMA,
        pltpu.SemaphoreType.DMA,
    ],
    compiler_params=pltpu.CompilerParams(
        use_tc_tiling_on_sc=False, needs_layout_passes=False),
)
def _gather(table_hbm, idx_hbm, out_hbm, idx_v, buf0, buf1, tb0, tb1,
            gsem0, gsem1, wsem0, wsem1):
    wid = lax.axis_index("s") * 2 + lax.axis_index("c")
    base_t = wid * _CPW
    pltpu.sync_copy(idx_hbm.at[pl.ds(base_t, _CPW)], idx_v)

    iota = lax.iota(jnp.int32, 16)

    def transpose_into(buf, tb):
        def col(j, carry):
            cols = jnp.broadcast_to(j, (16,))
            jh = j // 8
            jl = j % 8
            for v in range(_KH * 8):
                kh = v // 8
                tv = v % 8
                rows = (kh * 128 + tv * 16) + iota
                val = plsc.load_gather(buf, [rows, cols])
                tb[(jh * _KH + kh) * 8 + jl, pl.ds(tv * 16, 16)] = val
            return carry

        lax.fori_loop(0, _D, col, 0)

    def fire_gather(t, buf, sem):
        pltpu.async_copy(table_hbm.at[idx_v.at[t]], buf, sem)

    def wait_gather(t, buf, sem):
        pltpu.make_async_copy(table_hbm.at[idx_v.at[t]], buf, sem).wait()

    def out_slice(t, jh):
        kh0 = (base_t + t) * _KH
        return out_hbm.at[pl.ds((jh * _NKH + kh0) * 8, _KH * 8)]

    def fire_wb(t, tb, sem):
        for jh in range(8):
            pltpu.async_copy(tb.at[pl.ds(jh * _KH * 8, _KH * 8)],
                             out_slice(t, jh), sem)

    def wait_wb(t, tb, sem):
        for jh in range(8):
            pltpu.make_async_copy(tb.at[pl.ds(jh * _KH * 8, _KH * 8)],
                                  out_slice(t, jh), sem).wait()

    fire_gather(0, buf0, gsem0)

    def pair(p, carry):
        t0 = 2 * p
        t1 = t0 + 1
        wait_gather(t0, buf0, gsem0)
        fire_gather(t1, buf1, gsem1)

        @pl.when(p > 0)
        def _():
            wait_wb(t0 - 1, tb1, wsem1)

        transpose_into(buf0, tb0)
        fire_wb(t0, tb0, wsem0)
        wait_gather(t1, buf1, gsem1)

        @pl.when(p < _PAIRS - 1)
        def _():
            fire_gather(t1 + 1, buf0, gsem0)

        wait_wb(t0, tb0, wsem0)
        transpose_into(buf1, tb1)
        fire_wb(t1, tb1, wsem1)
        return carry

    lax.fori_loop(0, _PAIRS, pair, 0)
    wait_wb(_CPW - 1, tb1, wsem1)


def kernel(x, weight):
    wt = jnp.pad(weight, ((0, 0), (0, 64))).reshape(2 * _TABLE_ROWS, _D)
    idx = jnp.reshape(x * 2, (_CPW * _NW, _T))
    o2 = _gather(wt, idx)
    o4 = o2.reshape(8, _NKH, 8, 128)
    return o4.transpose(1, 3, 0, 2).reshape(_N_ROWS, _D)


# speedup vs baseline: 2.1129x; 2.1129x over previous
import functools

import jax
import jax.numpy as jnp
from jax import lax
from jax.experimental import pallas as pl
from jax.experimental.pallas import tpu as pltpu
from jax.experimental.pallas import tpu_sc as plsc

_N_ROWS = 819200
_TABLE_ROWS = 1000000
_D = 64
_NW = 32
_CHUNK = 128
_KC = 4
_ROWS_PER_DMA = _KC * _CHUNK
_CPW = _N_ROWS // (_NW * _ROWS_PER_DMA)

_mesh = plsc.VectorSubcoreMesh(core_axis_name="c", subcore_axis_name="s")


@functools.partial(
    pl.kernel,
    out_type=jax.ShapeDtypeStruct((_N_ROWS, 128), jnp.float32),
    mesh=_mesh,
    scratch_types=[
        pltpu.VMEM((_CPW, _ROWS_PER_DMA), jnp.int32),
        pltpu.VMEM((_ROWS_PER_DMA, _D), jnp.float32),
        pltpu.VMEM((_ROWS_PER_DMA, _D), jnp.float32),
        pltpu.SemaphoreType.DMA,
        pltpu.SemaphoreType.DMA,
        pltpu.SemaphoreType.DMA,
        pltpu.SemaphoreType.DMA,
    ],
    compiler_params=pltpu.CompilerParams(use_tc_tiling_on_sc=False),
)
def _gather(table_hbm, idx_hbm, out_hbm, idx_v, buf0, buf1,
            gsem0, gsem1, wsem0, wsem1):
    wid = lax.axis_index("s") * 2 + lax.axis_index("c")
    base = wid * _CPW
    pltpu.sync_copy(idx_hbm.at[pl.ds(base, _CPW)], idx_v)

    def out_slice(t):
        return out_hbm.at[pl.ds((base + t) * _ROWS_PER_DMA, _ROWS_PER_DMA),
                          pl.ds(0, _D)]

    def gather(t, buf, sem):
        return pltpu.async_copy(table_hbm.at[idx_v.at[t]], buf, sem)

    def wb(t, buf, sem):
        return pltpu.async_copy(
            buf, out_slice(t), sem)

    gather(0, buf0, gsem0)

    def pair(p, carry):
        t0 = 2 * p
        t1 = t0 + 1
        pltpu.make_async_copy(table_hbm.at[idx_v.at[t0]], buf0, gsem0).wait()

        @pl.when(p > 0)
        def _():
            pltpu.make_async_copy(
                buf1, out_slice(t0 - 1),
                wsem1).wait()

        gather(t1, buf1, gsem1)
        wb(t0, buf0, wsem0)
        pltpu.make_async_copy(table_hbm.at[idx_v.at[t1]], buf1, gsem1).wait()
        pltpu.make_async_copy(
            buf0, out_slice(t0), wsem0).wait()

        @pl.when(p < _CPW // 2 - 1)
        def _():
            gather(t1 + 1, buf0, gsem0)

        wb(t1, buf1, wsem1)
        return carry

    lax.fori_loop(0, _CPW // 2, pair, 0)
    pltpu.make_async_copy(
        buf1, out_slice(_CPW - 1), wsem1).wait()


def kernel(x, weight):
    wt = jnp.pad(weight, ((0, 0), (0, 64))).reshape(2 * _TABLE_ROWS, _D)
    idx = jnp.reshape(x * 2, (_N_ROWS // _ROWS_PER_DMA, _ROWS_PER_DMA))
    o = _gather(wt, idx)
    return o[:, :_D]
